# baseline (speedup 1.0000x reference)
import jax
import jax.numpy as jnp
from jax import lax
from jax.experimental import pallas as pl
from jax.experimental.pallas import tpu as pltpu, tpu_sc as plsc

RHO = 70.0
T, N = 1025, 20000
NPOS = 65536
SPAIRS = 100
TM1 = T - 1
NW = 32
PP = NPOS // NW
RT = TM1 // NW
SS = RT * SPAIRS
L = 16
SSP = SS + L
NVR = (SPAIRS + L - 1) // L
NRV = N // L


def _sigmoid(z):
    return 1.0 / (1.0 + jnp.exp(-z))


def _body(x2_hbm, th_hbm, tp_hbm, up_hbm, vp_hbm, us_hbm, vs_hbm,
          kpos_hbm, kneg_hbm,
          th_v, tp_v, up_v, vp_v, iu_v, iv_v, fu_v, fv_v, pg_v,
          gu_v, gv_v, su_v, sv_v, cb0, cb1, op_v, on_v, psem, rsem):
    wid = lax.axis_index("s") * 2 + lax.axis_index("c")
    xr = x2_hbm

    with jax.named_scope("inputs"):
        pltpu.sync_copy(th_hbm, th_v)
        eps = _sigmoid(th_v[...]) * 0.5
        pltpu.sync_copy(us_hbm.at[pl.ds(wid * SS, SS)], su_v.at[pl.ds(0, SS)])
        pltpu.sync_copy(vs_hbm.at[pl.ds(wid * SS, SS)], sv_v.at[pl.ds(0, SS)])
        base = wid * PP
        pltpu.sync_copy(tp_hbm.at[pl.ds(base, PP)], tp_v)
        pltpu.sync_copy(up_hbm.at[pl.ds(base, PP)], up_v)
        pltpu.sync_copy(vp_hbm.at[pl.ds(base, PP)], vp_v)

    row0 = wid * RT
    pltpu.async_copy(xr.at[pl.ds(row0 * NRV, NRV)], cb0, rsem)
    pltpu.async_copy(xr.at[pl.ds((row0 + 1) * NRV, NRV)], cb1, rsem)

    def pos_idx(k, c):
        sl = pl.ds(k * L, L)
        fu = tp_v[sl] * N + up_v[sl]
        fv = tp_v[sl] * N + vp_v[sl]
        iu_v[sl] = lax.shift_right_logical(fu, 4)
        iv_v[sl] = lax.shift_right_logical(fv, 4)
        fu_v[sl] = fu & (L - 1)
        fv_v[sl] = fv & (L - 1)
        return c
    lax.fori_loop(0, PP // L, pos_idx, 0)

    pltpu.async_copy(xr.at[iu_v], pg_v, psem)

    iota = lax.iota(jnp.int32, L)

    def row_kappa(cb, rr):
        def one_vec(q, acc):
            sl = pl.ds(rr * SPAIRS + q * L, L)
            cu_ = jnp.minimum(jnp.maximum(su_v[sl], 0), N - 1)
            cv_ = jnp.minimum(jnp.maximum(sv_v[sl], 0), N - 1)
            xu = plsc.load_gather(cb, [lax.shift_right_logical(cu_, 4),
                                       cu_ & (L - 1)])
            xv = plsc.load_gather(cb, [lax.shift_right_logical(cv_, 4),
                                       cv_ & (L - 1)])
            kap = _sigmoid(RHO * (eps - jnp.abs(xu - xv)))
            nvalid = SPAIRS - q * L
            return acc + jnp.where(iota < nvalid, kap, 0.0)
        acc = lax.fori_loop(0, NVR, one_vec, jnp.zeros((L,), jnp.float32))
        return jnp.sum(acc)

    def lane_acc(rr, s):
        return jnp.where(iota == (rr & (L - 1)), 1.0 - s * (1.0 / SPAIRS), 0.0)

    def pair_body(i, carry):
        on_a, on_b = carry
        r0 = 2 * i
        r1 = 2 * i + 1

        pltpu.make_async_copy(xr.at[pl.ds(0, NRV)], cb0, rsem).wait()
        s0 = row_kappa(cb0, r0)
        nxt0 = row0 + jnp.minimum(r0 + 2, RT - 1)
        pltpu.async_copy(xr.at[pl.ds(nxt0 * NRV, NRV)], cb0, rsem)

        pltpu.make_async_copy(xr.at[pl.ds(0, NRV)], cb1, rsem).wait()
        s1 = row_kappa(cb1, r1)
        nxt1 = row0 + jnp.minimum(r1 + 2, RT - 1)
        pltpu.async_copy(xr.at[pl.ds(nxt1 * NRV, NRV)], cb1, rsem)

        both = lane_acc(r0, s0) + lane_acc(r1, s1)
        in_a = jnp.where(r0 < L, both, 0.0)
        return (on_a + in_a, on_b + (both - in_a))

    def extract(pg, fe, dst):
        def one(k, c):
            sl = pl.ds(k * L, L)
            dst[sl] = plsc.load_gather(pg, [k * L + iota, fe[sl]])
            return c
        lax.fori_loop(0, PP // L, one, 0)

    zero = jnp.zeros((L,), jnp.float32)
    with jax.named_scope("sample_loop_a"):
        on_a, on_b = lax.fori_loop(0, RT // 4, pair_body, (zero, zero))

    with jax.named_scope("u_extract"):
        pltpu.make_async_copy(xr.at[pl.ds(0, PP)], pg_v, psem).wait()
        extract(pg_v, fu_v, gu_v)
        pltpu.async_copy(xr.at[iv_v], pg_v, psem)

    with jax.named_scope("sample_loop_b"):
        on_a, on_b = lax.fori_loop(RT // 4, RT // 2, pair_body, (on_a, on_b))
        pltpu.make_async_copy(xr.at[pl.ds(0, NRV)], cb0, rsem).wait()
        pltpu.make_async_copy(xr.at[pl.ds(0, NRV)], cb1, rsem).wait()

    on_v[pl.ds(0, L)] = on_a
    on_v[pl.ds(L, L)] = on_b
    pltpu.sync_copy(on_v, kneg_hbm.at[pl.ds(wid * RT, RT)])

    with jax.named_scope("v_extract"):
        pltpu.make_async_copy(xr.at[pl.ds(0, PP)], pg_v, psem).wait()
        extract(pg_v, fv_v, gv_v)

    def pos_kap(k, c):
        sl = pl.ds(k * L, L)
        d = gu_v[sl] - gv_v[sl]
        op_v[sl] = _sigmoid(RHO * (eps - jnp.abs(d)))
        return c
    with jax.named_scope("pos_compute"):
        lax.fori_loop(0, PP // L, pos_kap, 0)
        pltpu.sync_copy(op_v, kpos_hbm.at[pl.ds(base, PP)])


def kernel(X, theta, u_pos, v_pos, t_pos, u_sample, v_sample):
    xv = X.reshape(T * N // L, L)
    th16 = jnp.broadcast_to(theta.astype(jnp.float32), (L,))
    us_f = u_sample.reshape(-1)
    vs_f = v_sample.reshape(-1)

    mesh = plsc.VectorSubcoreMesh(core_axis_name="c", subcore_axis_name="s")
    run = pl.kernel(
        _body,
        out_type=(
            jax.ShapeDtypeStruct((NPOS,), jnp.float32),
            jax.ShapeDtypeStruct((TM1,), jnp.float32),
        ),
        mesh=mesh,
        compiler_params=pltpu.CompilerParams(
            use_tc_tiling_on_sc=False, needs_layout_passes=False),
        scratch_types=[
            pltpu.VMEM((L,), jnp.float32),
            pltpu.VMEM((PP,), jnp.int32),
            pltpu.VMEM((PP,), jnp.int32),
            pltpu.VMEM((PP,), jnp.int32),
            pltpu.VMEM((PP,), jnp.int32),
            pltpu.VMEM((PP,), jnp.int32),
            pltpu.VMEM((PP,), jnp.int32),
            pltpu.VMEM((PP,), jnp.int32),
            pltpu.VMEM((PP, L), jnp.float32),
            pltpu.VMEM((PP,), jnp.float32),
            pltpu.VMEM((PP,), jnp.float32),
            pltpu.VMEM((SSP,), jnp.int32),
            pltpu.VMEM((SSP,), jnp.int32),
            pltpu.VMEM((NRV, L), jnp.float32),
            pltpu.VMEM((NRV, L), jnp.float32),
            pltpu.VMEM((PP,), jnp.float32),
            pltpu.VMEM((RT,), jnp.float32),
            pltpu.SemaphoreType.DMA,
            pltpu.SemaphoreType.DMA,
        ],
    )
    kappa_pos, kappa_neg = run(xv, th16, t_pos, u_pos, v_pos, us_f, vs_f)
    return kappa_pos, kappa_neg


# speedup vs baseline: 1.1898x; 1.1898x over previous
import jax
import jax.numpy as jnp
from jax import lax
from jax.experimental import pallas as pl
from jax.experimental.pallas import tpu as pltpu, tpu_sc as plsc

RHO = 70.0
T, N = 1025, 20000
NPOS = 65536
SPAIRS = 100
TM1 = T - 1
NW = 32
PP = NPOS // NW
RT = TM1 // NW
SS = RT * SPAIRS
L = 16


def _sigmoid(z):
    return 1.0 / (1.0 + jnp.exp(-z))


def _body(x_hbm, th_hbm, tp_hbm, up_hbm, vp_hbm, us_hbm, vs_hbm,
          kpos_hbm, kneg_hbm,
          th_v, tp_v, up_v, vp_v, iu_v, iv_v, gu_v, gv_v,
          su_v, sv_v, siu_v, siv_v, sgu_v, sgv_v, op_v, on_v, sem):
    wid = lax.axis_index("s") * 2 + lax.axis_index("c")

    pltpu.sync_copy(th_hbm, th_v)
    eps = _sigmoid(th_v[...]) * 0.5

    base = wid * PP
    pltpu.sync_copy(tp_hbm.at[pl.ds(base, PP)], tp_v)
    pltpu.sync_copy(up_hbm.at[pl.ds(base, PP)], up_v)
    pltpu.sync_copy(vp_hbm.at[pl.ds(base, PP)], vp_v)
    pltpu.sync_copy(us_hbm.at[wid], su_v)
    pltpu.sync_copy(vs_hbm.at[wid], sv_v)

    def pos_idx(k, c):
        sl = pl.ds(k * L, L)
        roff = tp_v[sl] * N
        iu_v[sl] = roff + up_v[sl]
        iv_v[sl] = roff + vp_v[sl]
        return c
    lax.fori_loop(0, PP // L, pos_idx, 0)

    iota = lax.iota(jnp.int32, L)
    t0 = (wid * RT + iota) * N
    t1 = (wid * RT + L + iota) * N

    def samp_idx(j, c):
        b = j * 2 * L
        s0 = pl.ds(b, L)
        s1 = pl.ds(b + L, L)
        siu_v[s0] = su_v[s0] + t0
        siu_v[s1] = su_v[s1] + t1
        siv_v[s0] = sv_v[s0] + t0
        siv_v[s1] = sv_v[s1] + t1
        return c
    lax.fori_loop(0, SPAIRS, samp_idx, 0)

    cu = pltpu.async_copy(x_hbm.at[iu_v], gu_v, sem)
    cv = pltpu.async_copy(x_hbm.at[iv_v], gv_v, sem)
    gsu = pltpu.async_copy(x_hbm.at[siu_v], sgu_v, sem)
    gsv = pltpu.async_copy(x_hbm.at[siv_v], sgv_v, sem)

    cu.wait()
    cv.wait()

    def pos_kap(k, c):
        sl = pl.ds(k * L, L)
        d = gu_v[sl] - gv_v[sl]
        op_v[sl] = _sigmoid(RHO * (eps - jnp.abs(d)))
        return c
    lax.fori_loop(0, PP // L, pos_kap, 0)
    pltpu.sync_copy(op_v, kpos_hbm.at[pl.ds(base, PP)])

    gsu.wait()
    gsv.wait()

    def samp_kap(j, acc):
        a0, a1 = acc
        b = j * 2 * L
        s0 = pl.ds(b, L)
        s1 = pl.ds(b + L, L)
        d0 = sgu_v[s0] - sgv_v[s0]
        d1 = sgu_v[s1] - sgv_v[s1]
        a0 = a0 + _sigmoid(RHO * (eps - jnp.abs(d0)))
        a1 = a1 + _sigmoid(RHO * (eps - jnp.abs(d1)))
        return (a0, a1)
    zero = jnp.zeros((L,), jnp.float32)
    a0, a1 = lax.fori_loop(0, SPAIRS, samp_kap, (zero, zero))

    on_v[pl.ds(0, L)] = 1.0 - a0 * (1.0 / SPAIRS)
    on_v[pl.ds(L, L)] = 1.0 - a1 * (1.0 / SPAIRS)
    pltpu.sync_copy(on_v, kneg_hbm.at[pl.ds(wid * RT, RT)])


def kernel(X, theta, u_pos, v_pos, t_pos, u_sample, v_sample):
    x_flat = X.reshape(-1)
    th16 = jnp.broadcast_to(theta.astype(jnp.float32), (L,))
    us_p = u_sample.reshape(NW, RT, SPAIRS).transpose(0, 2, 1).reshape(NW, SS)
    vs_p = v_sample.reshape(NW, RT, SPAIRS).transpose(0, 2, 1).reshape(NW, SS)

    mesh = plsc.VectorSubcoreMesh(core_axis_name="c", subcore_axis_name="s")
    run = pl.kernel(
        _body,
        out_type=(
            jax.ShapeDtypeStruct((NPOS,), jnp.float32),
            jax.ShapeDtypeStruct((TM1,), jnp.float32),
        ),
        mesh=mesh,
        compiler_params=pltpu.CompilerParams(
            use_tc_tiling_on_sc=False, needs_layout_passes=False),
        scratch_types=[
            pltpu.VMEM((L,), jnp.float32),
            pltpu.VMEM((PP,), jnp.int32),
            pltpu.VMEM((PP,), jnp.int32),
            pltpu.VMEM((PP,), jnp.int32),
            pltpu.VMEM((PP,), jnp.int32),
            pltpu.VMEM((PP,), jnp.int32),
            pltpu.VMEM((PP,), jnp.float32),
            pltpu.VMEM((PP,), jnp.float32),
            pltpu.VMEM((SS,), jnp.int32),
            pltpu.VMEM((SS,), jnp.int32),
            pltpu.VMEM((SS,), jnp.int32),
            pltpu.VMEM((SS,), jnp.int32),
            pltpu.VMEM((SS,), jnp.float32),
            pltpu.VMEM((SS,), jnp.float32),
            pltpu.VMEM((PP,), jnp.float32),
            pltpu.VMEM((RT,), jnp.float32),
            pltpu.SemaphoreType.DMA,
        ],
    )
    kappa_pos, kappa_neg = run(x_flat, th16, t_pos, u_pos, v_pos, us_p, vs_p)
    return kappa_pos, kappa_neg
